# Initial kernel scaffold; baseline (speedup 1.0000x reference)
import jax
import jax.numpy as jnp
from jax.experimental import pallas as pl


def kernel(pair_i, pair_j, basis, p1, W_pp, W_pi, W_ii):
    raise NotImplementedError("write your pallas kernel here")


# speedup vs baseline: 5.3522x; 5.3522x over previous
import functools

import jax
import jax.numpy as jnp
from jax import lax
from jax.experimental import pallas as pl
from jax.experimental.pallas import tpu as pltpu
from jax.experimental.pallas import tpu_sc as plsc

NC = 2
NS = 16
C = 40


def _mm_body(x_ref, w_ref, o_ref):
    o_ref[...] = jnp.dot(x_ref[...], w_ref[...],
                         preferred_element_type=jnp.float32)


def _matmul(x, w, bm):
    m, d = x.shape
    return pl.pallas_call(
        _mm_body,
        grid=(m // bm,),
        in_specs=[pl.BlockSpec((bm, d), lambda i: (i, 0)),
                  pl.BlockSpec((d, d), lambda i: (0, 0))],
        out_specs=pl.BlockSpec((bm, d), lambda i: (i, 0)),
        out_shape=jax.ShapeDtypeStruct((m, d), jnp.float32),
    )(x, w)


def _q_body(x_ref, wpp_ref, wpi_ref, o_ref):
    h = jnp.dot(x_ref[...], wpp_ref[...], preferred_element_type=jnp.float32)
    o_ref[...] = jnp.dot(h, wpi_ref[...], preferred_element_type=jnp.float32)


def _q_table(p1, w_pp, w_pi, bm):
    n, d = p1.shape
    return pl.pallas_call(
        _q_body,
        grid=(n // bm,),
        in_specs=[pl.BlockSpec((bm, d), lambda i: (i, 0)),
                  pl.BlockSpec((d, d), lambda i: (0, 0)),
                  pl.BlockSpec((d, d), lambda i: (0, 0))],
        out_specs=pl.BlockSpec((bm, d), lambda i: (i, 0)),
        out_shape=jax.ShapeDtypeStruct((n, d), jnp.float32),
    )(p1, w_pp, w_pi)


def _fin_body(seg_ref, w_ref, o_ref):
    s = seg_ref[0, 0] + seg_ref[0, 1]
    o_ref[0] = jnp.dot(s, w_ref[...], preferred_element_type=jnp.float32)


def _finalize(seg, w_ii):
    ns, nc, rpt, d = seg.shape
    out = pl.pallas_call(
        _fin_body,
        grid=(ns,),
        in_specs=[pl.BlockSpec((1, nc, rpt, d), lambda i: (i, 0, 0, 0)),
                  pl.BlockSpec((d, d), lambda i: (0, 0))],
        out_specs=pl.BlockSpec((1, rpt, d), lambda i: (i, 0, 0)),
        out_shape=jax.ShapeDtypeStruct((ns, rpt, d), jnp.float32),
    )(seg, w_ii)
    return out.reshape(ns * rpt, d)


BLK = 25

def _edge_kernel(e, n, d):
    ept = e // (NC * NS)
    nchunk = ept // C
    assert nchunk % BLK == 0
    nblock = nchunk // BLK
    rpt = n // NS
    mesh = plsc.VectorSubcoreMesh(core_axis_name="c", subcore_axis_name="s")

    scratch = []
    for _ in range(5):
        scratch += [pltpu.VMEM((C,), jnp.int32),
                    pltpu.VMEM((C,), jnp.int32),
                    pltpu.VMEM((C, d), jnp.float32)]
    for _ in range(2):
        scratch += [pltpu.VMEM((C, d), jnp.float32),
                    pltpu.VMEM((C, d), jnp.float32)]
    scratch.append(pltpu.VMEM_SHARED((n, d), jnp.float32))
    scratch += [pltpu.SemaphoreType.DMA] * 4

    @functools.partial(
        pl.kernel, mesh=mesh,
        out_type=(jax.ShapeDtypeStruct((e, d), jnp.float32),
                  jax.ShapeDtypeStruct((NS, NC, rpt, d), jnp.float32)),
        scratch_types=scratch,
    )
    def body(pi_hbm, pj_hbm, b1_hbm, q_hbm, i1_hbm, seg_hbm, *refs):
        bufs = [refs[3 * b:3 * b + 3] for b in range(5)]
        qbufs = [refs[15 + 2 * b:15 + 2 * b + 2] for b in range(2)]
        acc_sh = refs[19]
        sem_a, sem_g0, sem_g1, sem_o = refs[20:24]
        sem_g = (sem_g0, sem_g1)
        cid = lax.axis_index("c")
        sid = lax.axis_index("s")
        wid = sid * NC + cid
        ebase = wid * ept

        def s1_issue(k0, b):
            ii, ij, ob = bufs[b % 5]
            base = ebase + (k0 + b) * C
            return (pltpu.async_copy(pi_hbm.at[pl.ds(base, C)], ii, sem_a),
                    pltpu.async_copy(pj_hbm.at[pl.ds(base, C)], ij, sem_a),
                    pltpu.async_copy(b1_hbm.at[pl.ds(base, C)], ob, sem_a))

        def g_issue(b):
            ii, ij, _ = bufs[b % 5]
            qi, qj = qbufs[b % 2]
            return (pltpu.async_copy(q_hbm.at[ii], qi, sem_g[b % 2]),
                    pltpu.async_copy(q_hbm.at[ij], qj, sem_g[b % 2]))

        def compute(b):
            _, _, ob = bufs[b % 5]
            qi, qj = qbufs[b % 2]

            def row(r, rc):
                for g in range(d // 16):
                    sl = pl.ds(g * 16, 16)
                    plsc.addupdate(ob.at[r, sl], qi[r, sl] + qj[r, sl])
                return rc
            lax.fori_loop(0, C, row, 0, unroll=2)

        zbuf = bufs[0][2]
        zero = jnp.zeros((16,), jnp.float32)

        def zrow(r, carry):
            for g in range(d // 16):
                zbuf[r, pl.ds(g * 16, 16)] = zero
            return carry
        lax.fori_loop(0, C, zrow, 0)
        done = 0
        while done < rpt:
            rows = min(C, rpt - done)
            pltpu.sync_copy(zbuf.at[pl.ds(0, rows)],
                            acc_sh.at[pl.ds(sid * rpt + done, rows)])
            done += rows
        plsc.subcore_barrier()

        def block(t, carry):
            k0 = t * BLK
            h_in = {}
            h_g = {}
            h_o = {}
            h_in[0] = s1_issue(k0, 0)
            for h in h_in[0]:
                h.wait()
            h_g[0] = g_issue(0)
            h_in[1] = s1_issue(k0, 1)
            for h in h_in[1]:
                h.wait()
            h_in[2] = s1_issue(k0, 2)
            for b in range(BLK):
                if b + 1 < BLK:
                    h_g[b + 1] = g_issue(b + 1)
                for h in h_g[b]:
                    h.wait()
                compute(b)
                ii, _, ob = bufs[b % 5]
                base = ebase + (k0 + b) * C
                pltpu.sync_copy(ob, acc_sh.at[ii], add=True)
                h_o[b] = pltpu.async_copy(
                    ob, i1_hbm.at[pl.ds(base, C)], sem_o)
                if b >= 1:
                    h_o.pop(b - 1).wait()
                if b + 3 < BLK:
                    for h in h_in.pop(b + 2):
                        h.wait()
                    h_in[b + 3] = s1_issue(k0, b + 3)
                elif b + 2 < BLK:
                    for h in h_in.pop(b + 2):
                        h.wait()
            h_o.pop(BLK - 1).wait()
            return carry
        lax.fori_loop(0, nblock, block, 0)

        plsc.subcore_barrier()
        pltpu.sync_copy(acc_sh.at[pl.ds(sid * rpt, rpt)],
                        seg_hbm.at[sid, cid])

    return body


def kernel(pair_i, pair_j, basis, p1, W_pp, W_pi, W_ii):
    e, d = basis.shape
    n = p1.shape[0]
    q = _q_table(p1, W_pp, W_pi, bm=2000)
    b1 = _matmul(basis, W_pi, bm=2000)
    i1, seg = _edge_kernel(e, n, d)(pair_i, pair_j, b1, q)
    p1_out = _finalize(seg, W_ii)
    return p1_out, i1


# speedup vs baseline: 5.3964x; 1.0083x over previous
import functools

import jax
import jax.numpy as jnp
from jax import lax
from jax.experimental import pallas as pl
from jax.experimental.pallas import tpu as pltpu
from jax.experimental.pallas import tpu_sc as plsc

NC = 2
NS = 16
C = 40


def _mm_body(x_ref, w_ref, o_ref):
    o_ref[...] = jnp.dot(x_ref[...], w_ref[...],
                         preferred_element_type=jnp.float32)


def _matmul(x, w, bm):
    m, d = x.shape
    return pl.pallas_call(
        _mm_body,
        grid=(m // bm,),
        in_specs=[pl.BlockSpec((bm, d), lambda i: (i, 0)),
                  pl.BlockSpec((d, d), lambda i: (0, 0))],
        out_specs=pl.BlockSpec((bm, d), lambda i: (i, 0)),
        out_shape=jax.ShapeDtypeStruct((m, d), jnp.float32),
    )(x, w)


def _q_body(x_ref, wpp_ref, wpi_ref, o_ref):
    h = jnp.dot(x_ref[...], wpp_ref[...], preferred_element_type=jnp.float32)
    o_ref[...] = jnp.dot(h, wpi_ref[...], preferred_element_type=jnp.float32)


def _q_table(p1, w_pp, w_pi, bm):
    n, d = p1.shape
    return pl.pallas_call(
        _q_body,
        grid=(n // bm,),
        in_specs=[pl.BlockSpec((bm, d), lambda i: (i, 0)),
                  pl.BlockSpec((d, d), lambda i: (0, 0)),
                  pl.BlockSpec((d, d), lambda i: (0, 0))],
        out_specs=pl.BlockSpec((bm, d), lambda i: (i, 0)),
        out_shape=jax.ShapeDtypeStruct((n, d), jnp.float32),
    )(p1, w_pp, w_pi)


def _fin_body(seg_ref, w_ref, o_ref):
    s = seg_ref[0, 0] + seg_ref[0, 1]
    o_ref[0] = jnp.dot(s, w_ref[...], preferred_element_type=jnp.float32)


def _finalize(seg, w_ii):
    ns, nc, rpt, d = seg.shape
    out = pl.pallas_call(
        _fin_body,
        grid=(ns,),
        in_specs=[pl.BlockSpec((1, nc, rpt, d), lambda i: (i, 0, 0, 0)),
                  pl.BlockSpec((d, d), lambda i: (0, 0))],
        out_specs=pl.BlockSpec((1, rpt, d), lambda i: (i, 0, 0)),
        out_shape=jax.ShapeDtypeStruct((ns, rpt, d), jnp.float32),
    )(seg, w_ii)
    return out.reshape(ns * rpt, d)


BLK = 25

def _edge_kernel(e, n, d):
    ept = e // (NC * NS)
    nchunk = ept // C
    assert nchunk % BLK == 0
    nblock = nchunk // BLK
    rpt = n // NS
    mesh = plsc.VectorSubcoreMesh(core_axis_name="c", subcore_axis_name="s")

    scratch = []
    for _ in range(5):
        scratch += [pltpu.VMEM((C,), jnp.int32),
                    pltpu.VMEM((C,), jnp.int32),
                    pltpu.VMEM((C, d), jnp.float32)]
    for _ in range(2):
        scratch += [pltpu.VMEM((C, d), jnp.float32),
                    pltpu.VMEM((C, d), jnp.float32)]
    scratch.append(pltpu.VMEM_SHARED((n, d), jnp.float32))
    scratch += [pltpu.SemaphoreType.DMA] * 5

    @functools.partial(
        pl.kernel, mesh=mesh,
        out_type=(jax.ShapeDtypeStruct((e, d), jnp.float32),
                  jax.ShapeDtypeStruct((NS, NC, rpt, d), jnp.float32)),
        scratch_types=scratch,
    )
    def body(pi_hbm, pj_hbm, b1_hbm, q_hbm, i1_hbm, seg_hbm, *refs):
        bufs = [refs[3 * b:3 * b + 3] for b in range(5)]
        qbufs = [refs[15 + 2 * b:15 + 2 * b + 2] for b in range(2)]
        acc_sh = refs[19]
        sem_a, sem_g0, sem_g1, sem_o, sem_s = refs[20:25]
        sem_g = (sem_g0, sem_g1)
        cid = lax.axis_index("c")
        sid = lax.axis_index("s")
        wid = sid * NC + cid
        ebase = wid * ept

        def s1_issue(k0, b):
            ii, ij, ob = bufs[b % 5]
            base = ebase + (k0 + b) * C
            return (pltpu.async_copy(pi_hbm.at[pl.ds(base, C)], ii, sem_a),
                    pltpu.async_copy(pj_hbm.at[pl.ds(base, C)], ij, sem_a),
                    pltpu.async_copy(b1_hbm.at[pl.ds(base, C)], ob, sem_a))

        def g_issue(b):
            ii, ij, _ = bufs[b % 5]
            qi, qj = qbufs[b % 2]
            return (pltpu.async_copy(q_hbm.at[ii], qi, sem_g[b % 2]),
                    pltpu.async_copy(q_hbm.at[ij], qj, sem_g[b % 2]))

        def compute(b):
            _, _, ob = bufs[b % 5]
            qi, qj = qbufs[b % 2]

            def row(r, rc):
                for g in range(d // 16):
                    sl = pl.ds(g * 16, 16)
                    plsc.addupdate(ob.at[r, sl], qi[r, sl] + qj[r, sl])
                return rc
            lax.fori_loop(0, C, row, 0, unroll=2)

        zbuf = bufs[0][2]
        zero = jnp.zeros((16,), jnp.float32)

        def zrow(r, carry):
            for g in range(d // 16):
                zbuf[r, pl.ds(g * 16, 16)] = zero
            return carry
        lax.fori_loop(0, C, zrow, 0)
        done = 0
        while done < rpt:
            rows = min(C, rpt - done)
            pltpu.sync_copy(zbuf.at[pl.ds(0, rows)],
                            acc_sh.at[pl.ds(sid * rpt + done, rows)])
            done += rows
        plsc.subcore_barrier()

        def block(t, carry):
            k0 = t * BLK
            h_in = {}
            h_g = {}
            h_o = {}
            h_s = {}
            h_in[0] = s1_issue(k0, 0)
            for h in h_in[0]:
                h.wait()
            h_g[0] = g_issue(0)
            h_in[1] = s1_issue(k0, 1)
            for h in h_in[1]:
                h.wait()
            h_in[2] = s1_issue(k0, 2)
            for b in range(BLK):
                if b + 1 < BLK:
                    h_g[b + 1] = g_issue(b + 1)
                for h in h_g[b]:
                    h.wait()
                compute(b)
                ii, _, ob = bufs[b % 5]
                base = ebase + (k0 + b) * C
                h_s[b] = pltpu.async_copy(ob, acc_sh.at[ii], sem_s, add=True)
                h_o[b] = pltpu.async_copy(
                    ob, i1_hbm.at[pl.ds(base, C)], sem_o)
                if b >= 1:
                    h_o.pop(b - 1).wait()
                    h_s.pop(b - 1).wait()
                if b + 3 < BLK:
                    for h in h_in.pop(b + 2):
                        h.wait()
                    h_in[b + 3] = s1_issue(k0, b + 3)
                elif b + 2 < BLK:
                    for h in h_in.pop(b + 2):
                        h.wait()
            h_o.pop(BLK - 1).wait()
            h_s.pop(BLK - 1).wait()
            return carry
        lax.fori_loop(0, nblock, block, 0)

        plsc.subcore_barrier()
        pltpu.sync_copy(acc_sh.at[pl.ds(sid * rpt, rpt)],
                        seg_hbm.at[sid, cid])

    return body


def kernel(pair_i, pair_j, basis, p1, W_pp, W_pi, W_ii):
    e, d = basis.shape
    n = p1.shape[0]
    q = _q_table(p1, W_pp, W_pi, bm=2000)
    b1 = _matmul(basis, W_pi, bm=2000)
    i1, seg = _edge_kernel(e, n, d)(pair_i, pair_j, b1, q)
    p1_out = _finalize(seg, W_ii)
    return p1_out, i1


# speedup vs baseline: 6.0140x; 1.1145x over previous
import functools

import jax
import jax.numpy as jnp
from jax import lax
from jax.experimental import pallas as pl
from jax.experimental.pallas import tpu as pltpu
from jax.experimental.pallas import tpu_sc as plsc

NC = 2
NS = 16
C = 40


def _mm_body(x_ref, w_ref, o_ref):
    o_ref[...] = jnp.dot(x_ref[...], w_ref[...],
                         preferred_element_type=jnp.float32)


def _matmul(x, w, bm):
    m, d = x.shape
    return pl.pallas_call(
        _mm_body,
        grid=(m // bm,),
        in_specs=[pl.BlockSpec((bm, d), lambda i: (i, 0)),
                  pl.BlockSpec((d, d), lambda i: (0, 0))],
        out_specs=pl.BlockSpec((bm, d), lambda i: (i, 0)),
        out_shape=jax.ShapeDtypeStruct((m, d), jnp.float32),
    )(x, w)


def _q_body(x_ref, wpp_ref, wpi_ref, o_ref):
    h = jnp.dot(x_ref[...], wpp_ref[...], preferred_element_type=jnp.float32)
    o_ref[...] = jnp.dot(h, wpi_ref[...], preferred_element_type=jnp.float32)


def _q_table(p1, w_pp, w_pi, bm):
    n, d = p1.shape
    return pl.pallas_call(
        _q_body,
        grid=(n // bm,),
        in_specs=[pl.BlockSpec((bm, d), lambda i: (i, 0)),
                  pl.BlockSpec((d, d), lambda i: (0, 0)),
                  pl.BlockSpec((d, d), lambda i: (0, 0))],
        out_specs=pl.BlockSpec((bm, d), lambda i: (i, 0)),
        out_shape=jax.ShapeDtypeStruct((n, d), jnp.float32),
    )(p1, w_pp, w_pi)


def _fin_body(seg_ref, w_ref, o_ref):
    s = seg_ref[0, 0] + seg_ref[0, 1]
    o_ref[0] = jnp.dot(s, w_ref[...], preferred_element_type=jnp.float32)


def _finalize(seg, w_ii):
    ns, nc, rpt, d = seg.shape
    out = pl.pallas_call(
        _fin_body,
        grid=(ns,),
        in_specs=[pl.BlockSpec((1, nc, rpt, d), lambda i: (i, 0, 0, 0)),
                  pl.BlockSpec((d, d), lambda i: (0, 0))],
        out_specs=pl.BlockSpec((1, rpt, d), lambda i: (i, 0, 0)),
        out_shape=jax.ShapeDtypeStruct((ns, rpt, d), jnp.float32),
    )(seg, w_ii)
    return out.reshape(ns * rpt, d)


BLK = 50

def _edge_kernel(e, n, d):
    ept = e // (NC * NS)
    nchunk = ept // C
    assert nchunk % BLK == 0
    nblock = nchunk // BLK
    rpt = n // NS
    mesh = plsc.VectorSubcoreMesh(core_axis_name="c", subcore_axis_name="s")

    scratch = []
    for _ in range(5):
        scratch += [pltpu.VMEM((C,), jnp.int32),
                    pltpu.VMEM((C,), jnp.int32),
                    pltpu.VMEM((C, d), jnp.float32)]
    for _ in range(2):
        scratch += [pltpu.VMEM((C, d), jnp.float32),
                    pltpu.VMEM((C, d), jnp.float32)]
    scratch.append(pltpu.VMEM_SHARED((n, d), jnp.float32))
    scratch += [pltpu.SemaphoreType.DMA] * 5

    @functools.partial(
        pl.kernel, mesh=mesh,
        out_type=(jax.ShapeDtypeStruct((e, d), jnp.float32),
                  jax.ShapeDtypeStruct((NS, NC, rpt, d), jnp.float32)),
        scratch_types=scratch,
    )
    def body(pi_hbm, pj_hbm, b1_hbm, q_hbm, i1_hbm, seg_hbm, *refs):
        bufs = [refs[3 * b:3 * b + 3] for b in range(5)]
        qbufs = [refs[15 + 2 * b:15 + 2 * b + 2] for b in range(2)]
        acc_sh = refs[19]
        sem_a, sem_g0, sem_g1, sem_o, sem_s = refs[20:25]
        sem_g = (sem_g0, sem_g1)
        cid = lax.axis_index("c")
        sid = lax.axis_index("s")
        wid = sid * NC + cid
        ebase = wid * ept

        def s1_issue(k0, b):
            ii, ij, ob = bufs[b % 5]
            base = ebase + (k0 + b) * C
            return (pltpu.async_copy(pi_hbm.at[pl.ds(base, C)], ii, sem_a),
                    pltpu.async_copy(pj_hbm.at[pl.ds(base, C)], ij, sem_a),
                    pltpu.async_copy(b1_hbm.at[pl.ds(base, C)], ob, sem_a))

        def g_issue(b):
            ii, ij, _ = bufs[b % 5]
            qi, qj = qbufs[b % 2]
            return (pltpu.async_copy(q_hbm.at[ii], qi, sem_g[b % 2]),
                    pltpu.async_copy(q_hbm.at[ij], qj, sem_g[b % 2]))

        def compute(b):
            _, _, ob = bufs[b % 5]
            qi, qj = qbufs[b % 2]

            def row(r, rc):
                for g in range(d // 16):
                    sl = pl.ds(g * 16, 16)
                    plsc.addupdate(ob.at[r, sl], qi[r, sl] + qj[r, sl])
                return rc
            lax.fori_loop(0, C, row, 0, unroll=2)

        zbuf = bufs[0][2]
        zero = jnp.zeros((16,), jnp.float32)

        def zrow(r, carry):
            for g in range(d // 16):
                zbuf[r, pl.ds(g * 16, 16)] = zero
            return carry
        lax.fori_loop(0, C, zrow, 0)
        done = 0
        while done < rpt:
            rows = min(C, rpt - done)
            pltpu.sync_copy(zbuf.at[pl.ds(0, rows)],
                            acc_sh.at[pl.ds(sid * rpt + done, rows)])
            done += rows
        plsc.subcore_barrier()

        def block(t, carry):
            k0 = t * BLK
            h_in = {}
            h_g = {}
            h_o = {}
            h_s = {}
            h_in[0] = s1_issue(k0, 0)
            for h in h_in[0]:
                h.wait()
            h_g[0] = g_issue(0)
            h_in[1] = s1_issue(k0, 1)
            for h in h_in[1]:
                h.wait()
            h_in[2] = s1_issue(k0, 2)
            for b in range(BLK):
                if b + 1 < BLK:
                    h_g[b + 1] = g_issue(b + 1)
                for h in h_g[b]:
                    h.wait()
                compute(b)
                ii, _, ob = bufs[b % 5]
                base = ebase + (k0 + b) * C
                h_s[b] = pltpu.async_copy(ob, acc_sh.at[ii], sem_s, add=True)
                h_o[b] = pltpu.async_copy(
                    ob, i1_hbm.at[pl.ds(base, C)], sem_o)
                if b >= 1:
                    h_o.pop(b - 1).wait()
                    h_s.pop(b - 1).wait()
                if b + 3 < BLK:
                    for h in h_in.pop(b + 2):
                        h.wait()
                    h_in[b + 3] = s1_issue(k0, b + 3)
                elif b + 2 < BLK:
                    for h in h_in.pop(b + 2):
                        h.wait()
            h_o.pop(BLK - 1).wait()
            h_s.pop(BLK - 1).wait()
            return carry
        lax.fori_loop(0, nblock, block, 0)

        plsc.subcore_barrier()
        pltpu.sync_copy(acc_sh.at[pl.ds(sid * rpt, rpt)],
                        seg_hbm.at[sid, cid])

    return body


def kernel(pair_i, pair_j, basis, p1, W_pp, W_pi, W_ii):
    e, d = basis.shape
    n = p1.shape[0]
    q = _q_table(p1, W_pp, W_pi, bm=2000)
    b1 = _matmul(basis, W_pi, bm=4000)
    i1, seg = _edge_kernel(e, n, d)(pair_i, pair_j, b1, q)
    p1_out = _finalize(seg, W_ii)
    return p1_out, i1


# speedup vs baseline: 6.2187x; 1.0340x over previous
import functools

import jax
import jax.numpy as jnp
from jax import lax
from jax.experimental import pallas as pl
from jax.experimental.pallas import tpu as pltpu
from jax.experimental.pallas import tpu_sc as plsc

NC = 2
NS = 16
C = 40


def _mm_body(x_ref, w_ref, o_ref):
    o_ref[...] = jnp.dot(x_ref[...], w_ref[...],
                         preferred_element_type=jnp.float32)


def _matmul(x, w, bm):
    m, d = x.shape
    return pl.pallas_call(
        _mm_body,
        grid=(m // bm,),
        in_specs=[pl.BlockSpec((bm, d), lambda i: (i, 0)),
                  pl.BlockSpec((d, d), lambda i: (0, 0))],
        out_specs=pl.BlockSpec((bm, d), lambda i: (i, 0)),
        out_shape=jax.ShapeDtypeStruct((m, d), jnp.float32),
    )(x, w)


def _q_body(x_ref, wpp_ref, wpi_ref, o_ref):
    h = jnp.dot(x_ref[...], wpp_ref[...], preferred_element_type=jnp.float32)
    o_ref[...] = jnp.dot(h, wpi_ref[...], preferred_element_type=jnp.float32)


def _q_table(p1, w_pp, w_pi, bm):
    n, d = p1.shape
    return pl.pallas_call(
        _q_body,
        grid=(n // bm,),
        in_specs=[pl.BlockSpec((bm, d), lambda i: (i, 0)),
                  pl.BlockSpec((d, d), lambda i: (0, 0)),
                  pl.BlockSpec((d, d), lambda i: (0, 0))],
        out_specs=pl.BlockSpec((bm, d), lambda i: (i, 0)),
        out_shape=jax.ShapeDtypeStruct((n, d), jnp.float32),
    )(p1, w_pp, w_pi)


def _fin_body(seg_ref, w_ref, o_ref):
    s = seg_ref[0, 0] + seg_ref[0, 1]
    o_ref[0] = jnp.dot(s, w_ref[...], preferred_element_type=jnp.float32)


def _finalize(seg, w_ii):
    ns, nc, rpt, d = seg.shape
    out = pl.pallas_call(
        _fin_body,
        grid=(ns,),
        in_specs=[pl.BlockSpec((1, nc, rpt, d), lambda i: (i, 0, 0, 0)),
                  pl.BlockSpec((d, d), lambda i: (0, 0))],
        out_specs=pl.BlockSpec((1, rpt, d), lambda i: (i, 0, 0)),
        out_shape=jax.ShapeDtypeStruct((ns, rpt, d), jnp.float32),
    )(seg, w_ii)
    return out.reshape(ns * rpt, d)


BLK = 50

def _edge_kernel(e, n, d):
    ept = e // (NC * NS)
    nchunk = ept // C
    assert nchunk % BLK == 0
    nblock = nchunk // BLK
    rpt = n // NS
    mesh = plsc.VectorSubcoreMesh(core_axis_name="c", subcore_axis_name="s")

    scratch = []
    for _ in range(5):
        scratch += [pltpu.VMEM((C,), jnp.int32),
                    pltpu.VMEM((C,), jnp.int32),
                    pltpu.VMEM((C, d), jnp.float32)]
    for _ in range(2):
        scratch += [pltpu.VMEM((C, d), jnp.float32),
                    pltpu.VMEM((C, d), jnp.float32)]
    scratch.append(pltpu.VMEM_SHARED((n, d), jnp.float32))
    scratch += [pltpu.SemaphoreType.DMA] * 5

    @functools.partial(
        pl.kernel, mesh=mesh,
        out_type=(jax.ShapeDtypeStruct((e, d), jnp.float32),
                  jax.ShapeDtypeStruct((NS, NC, rpt, d), jnp.float32)),
        scratch_types=scratch,
    )
    def body(pi_hbm, pj_hbm, b1_hbm, q_hbm, i1_hbm, seg_hbm, *refs):
        bufs = [refs[3 * b:3 * b + 3] for b in range(5)]
        qbufs = [refs[15 + 2 * b:15 + 2 * b + 2] for b in range(2)]
        acc_sh = refs[19]
        sem_a, sem_g0, sem_g1, sem_o, sem_s = refs[20:25]
        sem_g = (sem_g0, sem_g1)
        cid = lax.axis_index("c")
        sid = lax.axis_index("s")
        wid = sid * NC + cid
        ebase = wid * ept

        def s1_issue(k0, b):
            ii, ij, ob = bufs[b % 5]
            base = ebase + (k0 + b) * C
            return (pltpu.async_copy(pi_hbm.at[pl.ds(base, C)], ii, sem_a),
                    pltpu.async_copy(pj_hbm.at[pl.ds(base, C)], ij, sem_a),
                    pltpu.async_copy(b1_hbm.at[pl.ds(base, C)], ob, sem_a))

        def g_issue(b):
            ii, ij, _ = bufs[b % 5]
            qi, qj = qbufs[b % 2]
            return (pltpu.async_copy(q_hbm.at[ii], qi, sem_g[b % 2]),
                    pltpu.async_copy(q_hbm.at[ij], qj, sem_g[b % 2]))

        def compute(b):
            _, _, ob = bufs[b % 5]
            qi, qj = qbufs[b % 2]

            def row(r, rc):
                for g in range(d // 16):
                    sl = pl.ds(g * 16, 16)
                    plsc.addupdate(ob.at[r, sl], qi[r, sl] + qj[r, sl])
                return rc
            lax.fori_loop(0, C, row, 0, unroll=2)

        zbuf = bufs[0][2]
        zero = jnp.zeros((16,), jnp.float32)

        def zrow(r, carry):
            for g in range(d // 16):
                zbuf[r, pl.ds(g * 16, 16)] = zero
            return carry
        lax.fori_loop(0, C, zrow, 0)
        done = 0
        while done < rpt:
            rows = min(C, rpt - done)
            pltpu.sync_copy(zbuf.at[pl.ds(0, rows)],
                            acc_sh.at[pl.ds(sid * rpt + done, rows)])
            done += rows
        plsc.subcore_barrier()

        def block(t, carry):
            k0 = t * BLK
            h_in = {}
            h_g = {}
            h_o = {}
            h_s = {}
            h_in[0] = s1_issue(k0, 0)
            for h in h_in[0]:
                h.wait()
            h_g[0] = g_issue(0)
            h_in[1] = s1_issue(k0, 1)
            for h in h_in[1]:
                h.wait()
            h_in[2] = s1_issue(k0, 2)
            for b in range(BLK):
                if b + 1 < BLK:
                    h_g[b + 1] = g_issue(b + 1)
                for h in h_g[b]:
                    h.wait()
                compute(b)
                ii, _, ob = bufs[b % 5]
                base = ebase + (k0 + b) * C
                h_s[b] = pltpu.async_copy(ob, acc_sh.at[ii], sem_s, add=True)
                h_o[b] = pltpu.async_copy(
                    ob, i1_hbm.at[pl.ds(base, C)], sem_o)
                if b >= 1:
                    h_o.pop(b - 1).wait()
                    h_s.pop(b - 1).wait()
                if b + 3 < BLK:
                    for h in h_in.pop(b + 2):
                        h.wait()
                    h_in[b + 3] = s1_issue(k0, b + 3)
                elif b + 2 < BLK:
                    for h in h_in.pop(b + 2):
                        h.wait()
            h_o.pop(BLK - 1).wait()
            h_s.pop(BLK - 1).wait()
            return carry
        lax.fori_loop(0, nblock, block, 0)

        plsc.subcore_barrier()
        pltpu.sync_copy(acc_sh.at[pl.ds(sid * rpt, rpt)],
                        seg_hbm.at[sid, cid])

    return body


def kernel(pair_i, pair_j, basis, p1, W_pp, W_pi, W_ii):
    e, d = basis.shape
    n = p1.shape[0]
    q = _q_table(p1, W_pp, W_pi, bm=2000)
    b1 = _matmul(basis, W_pi, bm=8000)
    i1, seg = _edge_kernel(e, n, d)(pair_i, pair_j, b1, q)
    p1_out = _finalize(seg, W_ii)
    return p1_out, i1


# speedup vs baseline: 6.2506x; 1.0051x over previous
import functools

import jax
import jax.numpy as jnp
from jax import lax
from jax.experimental import pallas as pl
from jax.experimental.pallas import tpu as pltpu
from jax.experimental.pallas import tpu_sc as plsc

NC = 2
NS = 16
C = 40


def _mm_body(x_ref, w_ref, o_ref):
    o_ref[...] = jnp.dot(x_ref[...], w_ref[...],
                         preferred_element_type=jnp.float32)


def _matmul(x, w, bm):
    m, d = x.shape
    return pl.pallas_call(
        _mm_body,
        grid=(m // bm,),
        in_specs=[pl.BlockSpec((bm, d), lambda i: (i, 0)),
                  pl.BlockSpec((d, d), lambda i: (0, 0))],
        out_specs=pl.BlockSpec((bm, d), lambda i: (i, 0)),
        out_shape=jax.ShapeDtypeStruct((m, d), jnp.float32),
    )(x, w)


def _q_body(x_ref, wpp_ref, wpi_ref, o_ref):
    h = jnp.dot(x_ref[...], wpp_ref[...], preferred_element_type=jnp.float32)
    o_ref[...] = jnp.dot(h, wpi_ref[...], preferred_element_type=jnp.float32)


def _q_table(p1, w_pp, w_pi, bm):
    n, d = p1.shape
    return pl.pallas_call(
        _q_body,
        grid=(n // bm,),
        in_specs=[pl.BlockSpec((bm, d), lambda i: (i, 0)),
                  pl.BlockSpec((d, d), lambda i: (0, 0)),
                  pl.BlockSpec((d, d), lambda i: (0, 0))],
        out_specs=pl.BlockSpec((bm, d), lambda i: (i, 0)),
        out_shape=jax.ShapeDtypeStruct((n, d), jnp.float32),
    )(p1, w_pp, w_pi)


def _fin_body(seg_ref, w_ref, o_ref):
    s = seg_ref[0, 0] + seg_ref[0, 1]
    o_ref[0] = jnp.dot(s, w_ref[...], preferred_element_type=jnp.float32)


def _finalize(seg, w_ii):
    ns, nc, rpt, d = seg.shape
    out = pl.pallas_call(
        _fin_body,
        grid=(ns,),
        in_specs=[pl.BlockSpec((1, nc, rpt, d), lambda i: (i, 0, 0, 0)),
                  pl.BlockSpec((d, d), lambda i: (0, 0))],
        out_specs=pl.BlockSpec((1, rpt, d), lambda i: (i, 0, 0)),
        out_shape=jax.ShapeDtypeStruct((ns, rpt, d), jnp.float32),
    )(seg, w_ii)
    return out.reshape(ns * rpt, d)


BLK = 50

def _edge_kernel(e, n, d):
    ept = e // (NC * NS)
    nchunk = ept // C
    assert nchunk % BLK == 0
    nblock = nchunk // BLK
    rpt = n // NS
    mesh = plsc.VectorSubcoreMesh(core_axis_name="c", subcore_axis_name="s")

    scratch = []
    for _ in range(5):
        scratch += [pltpu.VMEM((C,), jnp.int32),
                    pltpu.VMEM((C,), jnp.int32),
                    pltpu.VMEM((C, d), jnp.float32)]
    for _ in range(2):
        scratch += [pltpu.VMEM((C, d), jnp.float32),
                    pltpu.VMEM((C, d), jnp.float32)]
    scratch.append(pltpu.VMEM_SHARED((n, d), jnp.float32))
    scratch += [pltpu.SemaphoreType.DMA] * 5

    @functools.partial(
        pl.kernel, mesh=mesh,
        out_type=(jax.ShapeDtypeStruct((e, d), jnp.float32),
                  jax.ShapeDtypeStruct((NS, NC, rpt, d), jnp.float32)),
        scratch_types=scratch,
    )
    def body(pi_hbm, pj_hbm, b1_hbm, q_hbm, i1_hbm, seg_hbm, *refs):
        bufs = [refs[3 * b:3 * b + 3] for b in range(5)]
        qbufs = [refs[15 + 2 * b:15 + 2 * b + 2] for b in range(2)]
        acc_sh = refs[19]
        sem_a, sem_g0, sem_g1, sem_o, sem_s = refs[20:25]
        sem_g = (sem_g0, sem_g1)
        cid = lax.axis_index("c")
        sid = lax.axis_index("s")
        wid = sid * NC + cid
        ebase = wid * ept

        def s1_issue(k0, b):
            ii, ij, ob = bufs[b % 5]
            base = ebase + (k0 + b) * C
            return (pltpu.async_copy(pi_hbm.at[pl.ds(base, C)], ii, sem_a),
                    pltpu.async_copy(pj_hbm.at[pl.ds(base, C)], ij, sem_a),
                    pltpu.async_copy(b1_hbm.at[pl.ds(base, C)], ob, sem_a))

        def g_issue(b):
            ii, ij, _ = bufs[b % 5]
            qi, qj = qbufs[b % 2]
            return (pltpu.async_copy(q_hbm.at[ii], qi, sem_g[b % 2]),
                    pltpu.async_copy(q_hbm.at[ij], qj, sem_g[b % 2]))

        def compute(b):
            _, _, ob = bufs[b % 5]
            qi, qj = qbufs[b % 2]

            def row(r, rc):
                for g in range(d // 16):
                    sl = pl.ds(g * 16, 16)
                    plsc.addupdate(ob.at[r, sl], qi[r, sl] + qj[r, sl])
                return rc
            lax.fori_loop(0, C, row, 0, unroll=2)

        zbuf = bufs[0][2]
        zero = jnp.zeros((16,), jnp.float32)

        def zrow(r, carry):
            for g in range(d // 16):
                zbuf[r, pl.ds(g * 16, 16)] = zero
            return carry
        lax.fori_loop(0, C, zrow, 0)
        done = 0
        while done < rpt:
            rows = min(C, rpt - done)
            pltpu.sync_copy(zbuf.at[pl.ds(0, rows)],
                            acc_sh.at[pl.ds(sid * rpt + done, rows)])
            done += rows
        plsc.subcore_barrier()

        def block(t, carry):
            k0 = t * BLK
            h_in = {}
            h_g = {}
            h_o = {}
            h_s = {}
            h_in[0] = s1_issue(k0, 0)
            for h in h_in[0]:
                h.wait()
            h_g[0] = g_issue(0)
            h_in[1] = s1_issue(k0, 1)
            for h in h_in[1]:
                h.wait()
            h_in[2] = s1_issue(k0, 2)
            for b in range(BLK):
                if b + 1 < BLK:
                    h_g[b + 1] = g_issue(b + 1)
                for h in h_g[b]:
                    h.wait()
                compute(b)
                ii, _, ob = bufs[b % 5]
                base = ebase + (k0 + b) * C
                h_s[b] = pltpu.async_copy(ob, acc_sh.at[ii], sem_s, add=True)
                h_o[b] = pltpu.async_copy(
                    ob, i1_hbm.at[pl.ds(base, C)], sem_o)
                if b >= 1:
                    h_o.pop(b - 1).wait()
                    h_s.pop(b - 1).wait()
                if b + 3 < BLK:
                    for h in h_in.pop(b + 2):
                        h.wait()
                    h_in[b + 3] = s1_issue(k0, b + 3)
                elif b + 2 < BLK:
                    for h in h_in.pop(b + 2):
                        h.wait()
            h_o.pop(BLK - 1).wait()
            h_s.pop(BLK - 1).wait()
            return carry
        lax.fori_loop(0, nblock, block, 0)

        plsc.subcore_barrier()
        pltpu.sync_copy(acc_sh.at[pl.ds(sid * rpt, rpt)],
                        seg_hbm.at[sid, cid])

    return body


def kernel(pair_i, pair_j, basis, p1, W_pp, W_pi, W_ii):
    e, d = basis.shape
    n = p1.shape[0]
    q = _q_table(p1, W_pp, W_pi, bm=2000)
    b1 = _matmul(basis, W_pi, bm=16000)
    i1, seg = _edge_kernel(e, n, d)(pair_i, pair_j, b1, q)
    p1_out = _finalize(seg, W_ii)
    return p1_out, i1
